# Initial kernel scaffold; baseline (speedup 1.0000x reference)
import jax
import jax.numpy as jnp
from jax.experimental import pallas as pl


def kernel(xyz1, xyz2):
    raise NotImplementedError("write your pallas kernel here")


# speedup vs baseline: 1.1185x; 1.1185x over previous
import functools

import jax
import jax.numpy as jnp
from jax.experimental import pallas as pl
from jax.experimental.pallas import tpu as pltpu

_WEIGHT = 0.6
_MCHUNK = 1024


def _chamfer_body(a_ref, bt_ref, out_ref, rmin_ref, acc_ref, *, rscale, cscale):
    b = pl.program_id(0)
    mi = pl.program_id(1)
    nb = pl.num_programs(0)
    nm = pl.num_programs(1)

    a = a_ref[0]
    bt = bt_ref[0]
    ab = jax.lax.dot_general(
        a, bt, (((1,), (0,)), ((), ())),
        preferred_element_type=jnp.float32)
    aa = jnp.sum(a * a, axis=1, keepdims=True)
    bb = jnp.sum(bt * bt, axis=0, keepdims=True)
    d = aa + bb - 2.0 * ab

    cmin = jnp.maximum(jnp.min(d, axis=0), 0.0)
    csum = jnp.sum(cmin)

    rmin_chunk = jnp.min(d, axis=1, keepdims=True)

    @pl.when(mi == 0)
    def _():
        rmin_ref[...] = rmin_chunk

    @pl.when(mi != 0)
    def _():
        rmin_ref[...] = jnp.minimum(rmin_ref[...], rmin_chunk)

    @pl.when(jnp.logical_and(b == 0, mi == 0))
    def _():
        acc_ref[0] = 0.0

    acc_ref[0] += csum * cscale

    @pl.when(mi == nm - 1)
    def _():
        rsum = jnp.sum(jnp.maximum(rmin_ref[...], 0.0))
        acc_ref[0] += rsum * rscale

    @pl.when(jnp.logical_and(b == nb - 1, mi == nm - 1))
    def _():
        out_ref[0, 0] = acc_ref[0]


def kernel(xyz1, xyz2):
    B, N, _ = xyz1.shape
    M = xyz2.shape[1]
    f32 = jnp.float32

    a_pad = jnp.concatenate([xyz1, jnp.zeros((B, N, 5), f32)], axis=-1)
    b_pad = jnp.concatenate([xyz2, jnp.zeros((B, M, 5), f32)], axis=-1)
    bt_pad = jnp.swapaxes(b_pad, 1, 2)

    nm = M // _MCHUNK
    rscale = 0.5 * _WEIGHT / (B * N)
    cscale = 0.5 * _WEIGHT / (B * M)

    out = pl.pallas_call(
        functools.partial(_chamfer_body, rscale=rscale, cscale=cscale),
        grid=(B, nm),
        in_specs=[
            pl.BlockSpec((1, N, 8), lambda b, mi: (b, 0, 0)),
            pl.BlockSpec((1, 8, _MCHUNK), lambda b, mi: (b, 0, mi)),
        ],
        out_specs=pl.BlockSpec(memory_space=pltpu.SMEM),
        out_shape=jax.ShapeDtypeStruct((1, 1), f32),
        scratch_shapes=[
            pltpu.VMEM((N, 1), f32),
            pltpu.SMEM((1,), f32),
        ],
    )(a_pad, bt_pad)
    return out[0, 0]


# speedup vs baseline: 1.2325x; 1.1019x over previous
import functools

import jax
import jax.numpy as jnp
from jax.experimental import pallas as pl
from jax.experimental.pallas import tpu as pltpu

_WEIGHT = 0.6
_MCHUNK = 1024


def _chamfer_body(a_ref, bt_ref, out_ref, rmin_ref, acc_ref, *, rscale, cscale):
    b = pl.program_id(0)
    mi = pl.program_id(1)
    nb = pl.num_programs(0)
    nm = pl.num_programs(1)

    a = a_ref[0]
    bt = bt_ref[0]
    ab = jax.lax.dot_general(
        a, bt, (((1,), (0,)), ((), ())),
        preferred_element_type=jnp.float32)
    aa = jnp.sum(a * a, axis=1, keepdims=True)
    bb = 0.25 * jnp.sum(bt * bt, axis=0, keepdims=True)

    cmin = jnp.maximum(bb[0] + jnp.min(ab + aa, axis=0), 0.0)
    csum = jnp.sum(cmin)

    rmin_chunk = jnp.min(ab + bb, axis=1, keepdims=True)

    @pl.when(mi == 0)
    def _():
        rmin_ref[...] = rmin_chunk

    @pl.when(mi != 0)
    def _():
        rmin_ref[...] = jnp.minimum(rmin_ref[...], rmin_chunk)

    @pl.when(jnp.logical_and(b == 0, mi == 0))
    def _():
        acc_ref[0] = 0.0

    acc_ref[0] += csum * cscale

    @pl.when(mi == nm - 1)
    def _():
        rsum = jnp.sum(jnp.maximum(rmin_ref[...] + aa, 0.0))
        acc_ref[0] += rsum * rscale

    @pl.when(jnp.logical_and(b == nb - 1, mi == nm - 1))
    def _():
        out_ref[0, 0] = acc_ref[0]


def kernel(xyz1, xyz2):
    B, N, _ = xyz1.shape
    M = xyz2.shape[1]
    f32 = jnp.float32

    a_pad = jnp.concatenate([xyz1, jnp.zeros((B, N, 5), f32)], axis=-1)
    b_pad = jnp.concatenate([-2.0 * xyz2, jnp.zeros((B, M, 5), f32)], axis=-1)
    bt_pad = jnp.swapaxes(b_pad, 1, 2)

    nm = M // _MCHUNK
    rscale = 0.5 * _WEIGHT / (B * N)
    cscale = 0.5 * _WEIGHT / (B * M)

    out = pl.pallas_call(
        functools.partial(_chamfer_body, rscale=rscale, cscale=cscale),
        grid=(B, nm),
        in_specs=[
            pl.BlockSpec((1, N, 8), lambda b, mi: (b, 0, 0)),
            pl.BlockSpec((1, 8, _MCHUNK), lambda b, mi: (b, 0, mi)),
        ],
        out_specs=pl.BlockSpec(memory_space=pltpu.SMEM),
        out_shape=jax.ShapeDtypeStruct((1, 1), f32),
        scratch_shapes=[
            pltpu.VMEM((N, 1), f32),
            pltpu.SMEM((1,), f32),
        ],
    )(a_pad, bt_pad)
    return out[0, 0]


# speedup vs baseline: 1.3323x; 1.0810x over previous
import functools

import jax
import jax.numpy as jnp
from jax.experimental import pallas as pl
from jax.experimental.pallas import tpu as pltpu

_WEIGHT = 0.6
_MCHUNK = 2048


def _chamfer_body(a_ref, bt_ref, out_ref, rmin_ref, acc_ref, *, rscale, cscale):
    b = pl.program_id(0)
    mi = pl.program_id(1)
    nb = pl.num_programs(0)
    nm = pl.num_programs(1)

    a = a_ref[0]
    bt = bt_ref[0]
    ab = jax.lax.dot_general(
        a, bt, (((1,), (0,)), ((), ())),
        preferred_element_type=jnp.float32)
    aa = jnp.sum(a * a, axis=1, keepdims=True)
    bb = 0.25 * jnp.sum(bt * bt, axis=0, keepdims=True)

    cmin = jnp.maximum(bb[0] + jnp.min(ab + aa, axis=0), 0.0)
    csum = jnp.sum(cmin)

    rmin_chunk = jnp.min(ab + bb, axis=1, keepdims=True)

    @pl.when(mi == 0)
    def _():
        rmin_ref[...] = rmin_chunk

    @pl.when(mi != 0)
    def _():
        rmin_ref[...] = jnp.minimum(rmin_ref[...], rmin_chunk)

    @pl.when(jnp.logical_and(b == 0, mi == 0))
    def _():
        acc_ref[0] = 0.0

    acc_ref[0] += csum * cscale

    @pl.when(mi == nm - 1)
    def _():
        rsum = jnp.sum(jnp.maximum(rmin_ref[...] + aa, 0.0))
        acc_ref[0] += rsum * rscale

    @pl.when(jnp.logical_and(b == nb - 1, mi == nm - 1))
    def _():
        out_ref[0, 0] = acc_ref[0]


def kernel(xyz1, xyz2):
    B, N, _ = xyz1.shape
    M = xyz2.shape[1]
    f32 = jnp.float32

    a_pad = jnp.concatenate([xyz1, jnp.zeros((B, N, 5), f32)], axis=-1)
    b_pad = jnp.concatenate([-2.0 * xyz2, jnp.zeros((B, M, 5), f32)], axis=-1)
    bt_pad = jnp.swapaxes(b_pad, 1, 2)

    nm = M // _MCHUNK
    rscale = 0.5 * _WEIGHT / (B * N)
    cscale = 0.5 * _WEIGHT / (B * M)

    out = pl.pallas_call(
        functools.partial(_chamfer_body, rscale=rscale, cscale=cscale),
        grid=(B, nm),
        in_specs=[
            pl.BlockSpec((1, N, 8), lambda b, mi: (b, 0, 0)),
            pl.BlockSpec((1, 8, _MCHUNK), lambda b, mi: (b, 0, mi)),
        ],
        out_specs=pl.BlockSpec(memory_space=pltpu.SMEM),
        out_shape=jax.ShapeDtypeStruct((1, 1), f32),
        scratch_shapes=[
            pltpu.VMEM((N, 1), f32),
            pltpu.SMEM((1,), f32),
        ],
    )(a_pad, bt_pad)
    return out[0, 0]
